# baseline (speedup 1.0000x reference)
import functools

import jax
import jax.numpy as jnp
from jax import lax
from jax.experimental import pallas as pl
from jax.experimental.pallas import tpu as pltpu
from jax.experimental.pallas import tpu_sc as plsc

VOCAB = 100000
SEQ_LEN = 2048
EMBED = 128
BATCH = 4

NC = 2
NS = 16
NW = NC * NS
B_PER_W = (BATCH * SEQ_LEN) // NW
CH = 64
NCH = B_PER_W // CH
W_PER_B = SEQ_LEN // B_PER_W
LANES = 16
SCALE = 11.31370849898476


def _sc_embed(idx, token_table, pos_table):
  mesh = plsc.VectorSubcoreMesh(core_axis_name="c", subcore_axis_name="s")

  @functools.partial(
      pl.kernel,
      mesh=mesh,
      out_type=jax.ShapeDtypeStruct((BATCH, SEQ_LEN, EMBED), jnp.float32),
      scratch_types=[
          pltpu.VMEM((NCH, CH), jnp.int32),
          pltpu.VMEM((B_PER_W, EMBED), jnp.float32),
          pltpu.VMEM((B_PER_W, EMBED), jnp.float32),
          pltpu.VMEM_SHARED((SEQ_LEN // NC, EMBED), jnp.float32),
          pltpu.SemaphoreType.DMA((NCH,)),
          pltpu.SemaphoreType.DMA((NCH,)),
          pltpu.SemaphoreType.DMA((NCH,)),
          pltpu.SemaphoreType.DMA((NCH,)),
          pltpu.SemaphoreType.DMA,
      ],
  )
  def k(idx_hbm, tok_hbm, pos_hbm, out_hbm, idx_v, tok_v, pos_v, spos,
        isem, gsem, psem, wsem, ssem):
    sid = lax.axis_index("s")
    cid = lax.axis_index("c")
    b = sid // (NS // BATCH)
    sl = (sid % (NS // BATCH)) * B_PER_W
    s0 = cid * (SEQ_LEN // NC) + sl
    idx_copies = [
        pltpu.async_copy(idx_hbm.at[b, pl.ds(s0 + c * CH, CH)],
                         idx_v.at[c], isem.at[c])
        for c in range(NCH)
    ]
    srows = SEQ_LEN // NC // NS
    stage = pltpu.async_copy(
        pos_hbm.at[pl.ds(cid * (SEQ_LEN // NC) + sid * srows, srows)],
        spos.at[pl.ds(sid * srows, srows)],
        ssem,
    )
    gathers = []
    for c in range(NCH):
      idx_copies[c].wait()
      gathers.append(
          pltpu.async_copy(
              tok_hbm.at[idx_v.at[c]],
              tok_v.at[pl.ds(c * CH, CH)],
              gsem.at[c],
          ))
    stage.wait()
    plsc.subcore_barrier()
    pos_copies = [
        pltpu.async_copy(
            spos.at[pl.ds(sl + c * CH, CH)],
            pos_v.at[pl.ds(c * CH, CH)],
            psem.at[c],
        ) for c in range(NCH)
    ]

    writes = []
    for c in range(NCH):
      gathers[c].wait()
      pos_copies[c].wait()

      def row(r, carry):
        for j in range(EMBED // LANES):
          sl = (r, pl.ds(j * LANES, LANES))
          plsc.addupdate(pos_v.at[sl], tok_v[sl] * SCALE)
        return carry

      lax.fori_loop(c * CH, (c + 1) * CH, row, 0, unroll=2)
      writes.append(
          pltpu.async_copy(
              pos_v.at[pl.ds(c * CH, CH)],
              out_hbm.at[b].at[pl.ds(s0 + c * CH, CH)],
              wsem.at[c],
          ))
    for w in writes:
      w.wait()

  return k(idx, token_table, pos_table)


def kernel(inputs, token_table, pos_table):
  return _sc_embed(inputs.astype(jnp.int32), token_table, pos_table)


# speedup vs baseline: 1.0038x; 1.0038x over previous
import functools

import jax
import jax.numpy as jnp
from jax import lax
from jax.experimental import pallas as pl
from jax.experimental.pallas import tpu as pltpu
from jax.experimental.pallas import tpu_sc as plsc

VOCAB = 100000
SEQ_LEN = 2048
EMBED = 128
BATCH = 4

NC = 2
NS = 16
NW = NC * NS
B_PER_W = (BATCH * SEQ_LEN) // NW
CH = 64
NCH = B_PER_W // CH
W_PER_B = SEQ_LEN // B_PER_W
LANES = 16
SCALE = 11.31370849898476


def _sc_embed(idx, token_table, pos_table):
  mesh = plsc.VectorSubcoreMesh(core_axis_name="c", subcore_axis_name="s")

  @functools.partial(
      pl.kernel,
      mesh=mesh,
      out_type=jax.ShapeDtypeStruct((BATCH, SEQ_LEN, EMBED), jnp.float32),
      scratch_types=[
          pltpu.VMEM((NCH, CH), jnp.int32),
          pltpu.VMEM((B_PER_W, EMBED), jnp.float32),
          pltpu.VMEM((B_PER_W, EMBED), jnp.float32),
          pltpu.VMEM_SHARED((SEQ_LEN // NC, EMBED), jnp.float32),
          pltpu.SemaphoreType.DMA((NCH,)),
          pltpu.SemaphoreType.DMA((NCH,)),
          pltpu.SemaphoreType.DMA((NCH,)),
          pltpu.SemaphoreType.DMA((NCH,)),
          pltpu.SemaphoreType.DMA,
      ],
  )
  def k(idx_hbm, tok_hbm, pos_hbm, out_hbm, idx_v, tok_v, pos_v, spos,
        isem, gsem, psem, wsem, ssem):
    sid = lax.axis_index("s")
    cid = lax.axis_index("c")
    b = sid // (NS // BATCH)
    sl = (sid % (NS // BATCH)) * B_PER_W
    s0 = cid * (SEQ_LEN // NC) + sl
    srows = SEQ_LEN // NC // NS
    stage = pltpu.async_copy(
        pos_hbm.at[pl.ds(cid * (SEQ_LEN // NC) + sid * srows, srows)],
        spos.at[pl.ds(sid * srows, srows)],
        ssem,
    )
    idx_copies = [
        pltpu.async_copy(idx_hbm.at[b, pl.ds(s0 + c * CH, CH)],
                         idx_v.at[c], isem.at[c])
        for c in range(NCH)
    ]
    gathers = []
    for c in range(NCH):
      idx_copies[c].wait()
      gathers.append(
          pltpu.async_copy(
              tok_hbm.at[idx_v.at[c]],
              tok_v.at[pl.ds(c * CH, CH)],
              gsem.at[c],
          ))
    stage.wait()
    plsc.subcore_barrier()
    pos_copies = [
        pltpu.async_copy(
            spos.at[pl.ds(sl + c * CH, CH)],
            pos_v.at[pl.ds(c * CH, CH)],
            psem.at[c],
        ) for c in range(NCH)
    ]

    writes = []
    for c in range(NCH):
      gathers[c].wait()
      pos_copies[c].wait()

      def row(r, carry):
        for j in range(EMBED // LANES):
          sl = (r, pl.ds(j * LANES, LANES))
          plsc.addupdate(pos_v.at[sl], tok_v[sl] * SCALE)
        return carry

      lax.fori_loop(c * CH, (c + 1) * CH, row, 0, unroll=2)
      writes.append(
          pltpu.async_copy(
              pos_v.at[pl.ds(c * CH, CH)],
              out_hbm.at[b].at[pl.ds(s0 + c * CH, CH)],
              wsem.at[c],
          ))
    for w in writes:
      w.wait()

  return k(idx, token_table, pos_table)


def kernel(inputs, token_table, pos_table):
  return _sc_embed(inputs.astype(jnp.int32), token_table, pos_table)


# speedup vs baseline: 1.0051x; 1.0013x over previous
import functools

import jax
import jax.numpy as jnp
from jax import lax
from jax.experimental import pallas as pl
from jax.experimental.pallas import tpu as pltpu
from jax.experimental.pallas import tpu_sc as plsc

VOCAB = 100000
SEQ_LEN = 2048
EMBED = 128
BATCH = 4

NC = 2
NS = 16
NW = NC * NS
B_PER_W = (BATCH * SEQ_LEN) // NW
CH = 64
NCH = B_PER_W // CH
LANES = 16
SCALE = 11.31370849898476


def _sc_embed(idx, token_table, pos_table):
  mesh = plsc.VectorSubcoreMesh(core_axis_name="c", subcore_axis_name="s")

  @functools.partial(
      pl.kernel,
      mesh=mesh,
      out_type=jax.ShapeDtypeStruct((BATCH, SEQ_LEN, EMBED), jnp.float32),
      scratch_types=[
          pltpu.VMEM((NCH, CH), jnp.int32),
          pltpu.VMEM((B_PER_W, EMBED), jnp.float32),
          pltpu.VMEM((B_PER_W, EMBED), jnp.float32),
          pltpu.VMEM_SHARED((SEQ_LEN // NC, EMBED), jnp.float32),
          pltpu.SemaphoreType.DMA((NCH,)),
          pltpu.SemaphoreType.DMA((NCH,)),
          pltpu.SemaphoreType.DMA((NCH,)),
          pltpu.SemaphoreType.DMA((NCH,)),
          pltpu.SemaphoreType.DMA,
      ],
  )
  def k(idx_hbm, tok_hbm, pos_hbm, out_hbm, idx_v, tok_v, pos_v, spos,
        isem, gsem, psem, wsem, ssem):
    sid = lax.axis_index("s")
    cid = lax.axis_index("c")
    b = sid // (NS // BATCH)
    sl = (sid % (NS // BATCH)) * B_PER_W
    s0 = cid * (SEQ_LEN // NC) + sl
    srows = SEQ_LEN // NC // NS
    stage = pltpu.async_copy(
        pos_hbm.at[pl.ds(cid * (SEQ_LEN // NC) + sid * srows, srows)],
        spos.at[pl.ds(sid * srows, srows)],
        ssem,
    )
    idx_copies = [
        pltpu.async_copy(idx_hbm.at[b, pl.ds(s0 + c * CH, CH)],
                         idx_v.at[c], isem.at[c])
        for c in range(NCH)
    ]
    gathers = []
    for c in range(NCH):
      idx_copies[c].wait()
      gathers.append(
          pltpu.async_copy(
              tok_hbm.at[idx_v.at[c]],
              tok_v.at[pl.ds(c * CH, CH)],
              gsem.at[c],
          ))
    stage.wait()
    plsc.subcore_barrier()
    pos_copies = [
        pltpu.async_copy(
            spos.at[pl.ds(sl + c * CH, CH)],
            pos_v.at[pl.ds(c * CH, CH)],
            psem.at[c],
        ) for c in range(NCH)
    ]

    writes = []
    for c in range(NCH):
      gathers[c].wait()
      pos_copies[c].wait()

      def row(r, carry):
        for j in range(EMBED // LANES):
          t = (r, pl.ds(j * LANES, LANES))
          plsc.addupdate(pos_v.at[t], tok_v[t] * SCALE)
        return carry

      lax.fori_loop(c * CH, (c + 1) * CH, row, 0, unroll=2)
      writes.append(
          pltpu.async_copy(
              pos_v.at[pl.ds(c * CH, CH)],
              out_hbm.at[b].at[pl.ds(s0 + c * CH, CH)],
              wsem.at[c],
          ))
    for w in writes:
      w.wait()

  return k(idx, token_table, pos_table)


def kernel(inputs, token_table, pos_table):
  return _sc_embed(inputs.astype(jnp.int32), token_table, pos_table)
